# baseline (speedup 1.0000x reference)
import functools

import jax
import jax.numpy as jnp
from jax import lax
from jax.experimental import pallas as pl
from jax.experimental.pallas import tpu as pltpu
from jax.experimental.pallas import tpu_sc as plsc

N = 10000
E = 320000
D = 128
G = 64

NC = 2
NS = 16
NW = NC * NS
E_PER_W = E // NW
CHUNK = 80
STEPS = E_PER_W // CHUNK
CNT_W = 16

N_PAD = 10240
ROWS_PER_TILE = N_PAD // NS
ZCOPIES = ROWS_PER_TILE // CHUNK

RB = 1000
NB = N // RB

_mesh = plsc.VectorSubcoreMesh(core_axis_name="c", subcore_axis_name="s")


def _make_edge_kernel():
    out_type = [jax.ShapeDtypeStruct((NC, N_PAD, D), jnp.float32)]
    scratch = [
        pltpu.VMEM((E_PER_W + CHUNK,), jnp.int32),
        pltpu.VMEM((STEPS, CHUNK), jnp.int32),
        pltpu.VMEM((CHUNK, D), jnp.float32),
        pltpu.VMEM((CHUNK, D), jnp.float32),
        pltpu.VMEM_SHARED((N_PAD, D), jnp.float32),
        pltpu.SemaphoreType.DMA,
        pltpu.SemaphoreType.DMA,
        pltpu.SemaphoreType.DMA,
        pltpu.SemaphoreType.DMA,
    ]

    def body(h_hbm, src2_hbm, dst3_hbm, zrows_hbm, agg_out,
             src_big, dst_big, rows_a, rows_b, acc_sh,
             sem_a, sem_a2, sem_b, sem_b2):
        cid = lax.axis_index("c")
        sid = lax.axis_index("s")
        wid = cid * NS + sid
        r0 = sid * ROWS_PER_TILE

        pltpu.sync_copy(src2_hbm.at[wid], src_big)
        pltpu.sync_copy(dst3_hbm.at[wid], dst_big)

        HC = CHUNK // 2

        def gather(c, buf, sem1, sem2):
            o = pl.multiple_of(c * CHUNK, 8)
            pltpu.async_copy(h_hbm.at[src_big.at[pl.ds(o, HC)]],
                             buf.at[pl.ds(0, HC)], sem1)
            pltpu.async_copy(h_hbm.at[src_big.at[pl.ds(o + HC, HC)]],
                             buf.at[pl.ds(HC, HC)], sem2)

        def gwait(buf, sem1, sem2):
            pltpu.make_async_copy(zrows_hbm.at[pl.ds(0, HC)],
                                  buf.at[pl.ds(0, HC)], sem1).wait()
            pltpu.make_async_copy(zrows_hbm.at[pl.ds(0, HC)],
                                  buf.at[pl.ds(HC, HC)], sem2).wait()

        def scat(c, buf):
            pltpu.sync_copy(buf, acc_sh.at[dst_big.at[c]], add=True)

        gather(0, rows_a, sem_a, sem_a2)
        gather(1, rows_b, sem_b, sem_b2)

        for k in range(ZCOPIES):
            pltpu.sync_copy(zrows_hbm, acc_sh.at[pl.ds(r0 + k * CHUNK, CHUNK)])
        plsc.subcore_barrier()

        def step(j, carry):
            c = 2 * j
            gwait(rows_a, sem_a, sem_a2)
            scat(c, rows_a)
            gather(c + 2, rows_a, sem_a, sem_a2)
            gwait(rows_b, sem_b, sem_b2)
            scat(c + 1, rows_b)
            gather(c + 3, rows_b, sem_b, sem_b2)
            return carry

        lax.fori_loop(0, (STEPS - 1) // 2, step, 0)
        gwait(rows_a, sem_a, sem_a2)
        scat(STEPS - 1, rows_a)
        pltpu.make_async_copy(zrows_hbm.at[pl.ds(0, HC)],
                              rows_b.at[pl.ds(0, HC)], sem_b).wait()
        pltpu.make_async_copy(zrows_hbm.at[pl.ds(0, HC)],
                              rows_b.at[pl.ds(HC, HC)], sem_b2).wait()
        plsc.subcore_barrier()

        for k in range(ZCOPIES):
            row = r0 + k * CHUNK
            pltpu.sync_copy(acc_sh.at[pl.ds(row, CHUNK)],
                            agg_out.at[cid, pl.ds(row, CHUNK)])

    return pl.kernel(body, out_type=out_type, mesh=_mesh,
                     scratch_types=scratch)


def _make_cnt_kernel():
    out_type = [jax.ShapeDtypeStruct((NC * N_PAD,), jnp.float32)]
    scratch = [
        pltpu.VMEM((STEPS, CHUNK), jnp.int32),
        pltpu.VMEM((N_PAD,), jnp.float32),
        pltpu.VMEM((ROWS_PER_TILE,), jnp.float32),
        pltpu.VMEM((ROWS_PER_TILE,), jnp.float32),
        pltpu.VMEM_SHARED((NS * N_PAD,), jnp.float32),
    ]

    def body(dst3_hbm, cnt_out, dst_big, hist_v, ctmp_v, cacc_v, cnt_sh):
        cid = lax.axis_index("c")
        sid = lax.axis_index("s")
        wid = cid * NS + sid
        r0 = sid * ROWS_PER_TILE

        pltpu.sync_copy(dst3_hbm.at[wid], dst_big)

        def zb(j, carry):
            hist_v[pl.ds(j * 16, 16)] = jnp.zeros((16,), jnp.float32)
            return carry
        lax.fori_loop(0, N_PAD // 16, zb, 0)

        ones16 = jnp.ones((16,), jnp.float32)

        def hb(m, carry):
            for k in range(CHUNK // 16):
                plsc.addupdate_scatter(
                    hist_v, [dst_big[m, pl.ds(k * 16, 16)]], ones16)
            return carry

        lax.fori_loop(0, STEPS, hb, 0)
        pltpu.sync_copy(hist_v, cnt_sh.at[pl.ds(sid * N_PAD, N_PAD)])
        plsc.subcore_barrier()

        for s in range(NS):
            pltpu.sync_copy(cnt_sh.at[pl.ds(s * N_PAD + r0, ROWS_PER_TILE)],
                            ctmp_v)

            def cb(j, carry, first=(s == 0)):
                sl = pl.ds(j * 16, 16)
                if first:
                    cacc_v[sl] = ctmp_v[sl]
                else:
                    cacc_v[sl] = cacc_v[sl] + ctmp_v[sl]
                return carry

            lax.fori_loop(0, ROWS_PER_TILE // 16, cb, 0)
        pltpu.sync_copy(
            cacc_v, cnt_out.at[pl.ds(cid * N_PAD + r0, ROWS_PER_TILE)])

    return pl.kernel(body, out_type=out_type, mesh=_mesh,
                     scratch_types=scratch,
                     compiler_params=pltpu.CompilerParams(
                         needs_layout_passes=False))


_edge_kernel = _make_edge_kernel()
_cnt_kernel = _make_cnt_kernel()


def _mean_matmul(agg_ref, c0_ref, c1_ref, h_ref, wl_ref, bl_ref, wr_ref):
    a = agg_ref[0] + agg_ref[1]
    inv = 1.0 / jnp.maximum(c0_ref[...] + c1_ref[...], 1.0)
    out = (jnp.dot(a * inv, wl_ref[...], preferred_element_type=jnp.float32)
           + bl_ref[...][None, :]
           + jnp.dot(h_ref[...], wr_ref[...], preferred_element_type=jnp.float32))
    return jnp.maximum(out, 0.0)


def _dense_body(agg_ref, c0_ref, c1_ref, h_ref, wl_ref, bl_ref, wr_ref, out_ref):
    out_ref[...] = _mean_matmul(agg_ref, c0_ref, c1_ref, h_ref, wl_ref, bl_ref,
                                wr_ref)


_DENSE_SPECS = [
    pl.BlockSpec((NC, RB, D), lambda i: (0, i, 0)),
    pl.BlockSpec((RB, 1), lambda i: (i, 0)),
    pl.BlockSpec((RB, 1), lambda i: (i, 0)),
    pl.BlockSpec((RB, D), lambda i: (i, 0)),
    pl.BlockSpec((D, D), lambda i: (0, 0)),
    pl.BlockSpec((D,), lambda i: (0,)),
    pl.BlockSpec((D, D), lambda i: (0, 0)),
]


def _dense(agg2, c0, c1, h, Wl, bl, Wr):
    return pl.pallas_call(
        _dense_body,
        grid=(NB,),
        in_specs=_DENSE_SPECS,
        out_specs=pl.BlockSpec((RB, D), lambda i: (i, 0)),
        out_shape=jax.ShapeDtypeStruct((N, D), jnp.float32),
    )(agg2, c0, c1, h, Wl, bl, Wr)


def _final_body(agg_ref, c0_ref, c1_ref, h_ref, wl_ref, bl_ref, wr_ref, batch_ref,
                wc1_ref, bc1_ref, wc2_ref, bc2_ref, out_ref, pool, cntg):
    i = pl.program_id(0)

    @pl.when(i == 0)
    def _init():
        pool[...] = jnp.zeros_like(pool)
        cntg[...] = jnp.zeros_like(cntg)

    h3 = _mean_matmul(agg_ref, c0_ref, c1_ref, h_ref, wl_ref, bl_ref, wr_ref)
    b = batch_ref[0]
    oh = (lax.broadcasted_iota(jnp.int32, (G, RB), 0)
          == jnp.broadcast_to(b, (G, RB))).astype(jnp.float32)
    pool[...] += jnp.dot(oh, h3, preferred_element_type=jnp.float32,
                         precision=lax.Precision.HIGHEST)
    cntg[...] += jnp.broadcast_to(jnp.sum(oh, axis=1, keepdims=True), (G, D))

    @pl.when(i == NB - 1)
    def _fin():
        g = pool[...] / jnp.maximum(cntg[...], 1.0)
        z = jnp.maximum(
            jnp.dot(g, wc1_ref[...], preferred_element_type=jnp.float32)
            + bc1_ref[...][None, :], 0.0)
        out_ref[...] = (jnp.dot(z, wc2_ref[...], preferred_element_type=jnp.float32)
                        + bc2_ref[...][None, :])


def _dense_final(agg2, c0, c1, h, Wl, bl, Wr, batch3, Wc1, bc1, Wc2, bc2):
    return pl.pallas_call(
        _final_body,
        grid=(NB,),
        in_specs=_DENSE_SPECS + [
            pl.BlockSpec((1, 1, RB), lambda i: (i, 0, 0)),
            pl.BlockSpec((D, D), lambda i: (0, 0)),
            pl.BlockSpec((D,), lambda i: (0,)),
            pl.BlockSpec((D, 2), lambda i: (0, 0)),
            pl.BlockSpec((2,), lambda i: (0,)),
        ],
        out_specs=pl.BlockSpec((G, 2), lambda i: (0, 0)),
        out_shape=jax.ShapeDtypeStruct((G, 2), jnp.float32),
        scratch_shapes=[
            pltpu.VMEM((G, D), jnp.float32),
            pltpu.VMEM((G, D), jnp.float32),
        ],
    )(agg2, c0, c1, h, Wl, bl, Wr, batch3, Wc1, bc1, Wc2, bc2)


def kernel(x, edge_index, batch, Wl0, bl0, Wr0, Wl1, bl1, Wr1, Wl2, bl2, Wr2,
           Wc1, bc1, Wc2, bc2):
    src = jnp.pad(edge_index[0].reshape(NW, E_PER_W), ((0, 0), (0, CHUNK)))
    dst = edge_index[1].reshape(NW, STEPS, CHUNK)
    zrows = jnp.zeros((CHUNK, D), jnp.float32)

    (cnt_flat,) = _cnt_kernel(dst)
    (agg0,) = _edge_kernel(x, src, dst, zrows)
    c0 = cnt_flat[:N_PAD].reshape(N_PAD, 1)
    c1 = cnt_flat[N_PAD:].reshape(N_PAD, 1)
    h1 = _dense(agg0, c0, c1, x, Wl0, bl0, Wr0)
    (agg1,) = _edge_kernel(h1, src, dst, zrows)
    h2 = _dense(agg1, c0, c1, h1, Wl1, bl1, Wr1)
    (agg2,) = _edge_kernel(h2, src, dst, zrows)
    batch3 = batch.reshape(NB, 1, RB)
    return _dense_final(agg2, c0, c1, h2, Wl2, bl2, Wr2, batch3,
                        Wc1, bc1, Wc2, bc2)


# speedup vs baseline: 1.4335x; 1.4335x over previous
import functools

import jax
import jax.numpy as jnp
from jax import lax
from jax.experimental import pallas as pl
from jax.experimental.pallas import tpu as pltpu
from jax.experimental.pallas import tpu_sc as plsc

N = 10000
E = 320000
D = 128
G = 64

NC = 2
NS = 16
NW = NC * NS
E_PER_W = E // NW
CHUNK = 80
STEPS = E_PER_W // CHUNK
CNT_W = 16

N_PAD = 10240
ROWS_PER_TILE = N_PAD // NS
ZCOPIES = ROWS_PER_TILE // CHUNK

RB = 1000
NB = N // RB

_mesh = plsc.VectorSubcoreMesh(core_axis_name="c", subcore_axis_name="s")


def _make_edge_kernel():
    out_type = [jax.ShapeDtypeStruct((NC, N_PAD, D), jnp.float32)]
    scratch = [
        pltpu.VMEM((E_PER_W,), jnp.int32),
        pltpu.VMEM((STEPS, CHUNK), jnp.int32),
        pltpu.VMEM((CHUNK, D), jnp.float32),
        pltpu.VMEM((CHUNK, D), jnp.float32),
        pltpu.VMEM_SHARED((N_PAD, D), jnp.float32),
        pltpu.SemaphoreType.DMA,
        pltpu.SemaphoreType.DMA,
        pltpu.SemaphoreType.DMA,
        pltpu.SemaphoreType.DMA,
    ]

    def body(h_hbm, src2_hbm, dst3_hbm, zrows_hbm, agg_out,
             src_big, dst_big, rows_a, rows_b, acc_sh,
             sem_a, sem_a2, sem_b, sem_b2):
        cid = lax.axis_index("c")
        sid = lax.axis_index("s")
        wid = cid * NS + sid
        r0 = sid * ROWS_PER_TILE

        pltpu.sync_copy(src2_hbm.at[wid], src_big)
        pltpu.sync_copy(dst3_hbm.at[wid], dst_big)

        HC = CHUNK // 2

        def gather(c, buf, sem1, sem2):
            o = pl.multiple_of(c * CHUNK, 8)
            pltpu.async_copy(h_hbm.at[src_big.at[pl.ds(o, HC)]],
                             buf.at[pl.ds(0, HC)], sem1)
            pltpu.async_copy(h_hbm.at[src_big.at[pl.ds(o + HC, HC)]],
                             buf.at[pl.ds(HC, HC)], sem2)

        def gwait(buf, sem1, sem2):
            pltpu.make_async_copy(zrows_hbm.at[pl.ds(0, HC)],
                                  buf.at[pl.ds(0, HC)], sem1).wait()
            pltpu.make_async_copy(zrows_hbm.at[pl.ds(0, HC)],
                                  buf.at[pl.ds(HC, HC)], sem2).wait()

        def scat(c, buf):
            pltpu.sync_copy(buf, acc_sh.at[dst_big.at[c]], add=True)

        gather(0, rows_a, sem_a, sem_a2)

        pltpu.sync_copy(zrows_hbm, rows_b)
        for k in range(ZCOPIES):
            pltpu.sync_copy(rows_b, acc_sh.at[pl.ds(r0 + k * CHUNK, CHUNK)])
        plsc.subcore_barrier()

        def step(j, carry):
            c = 2 * j
            gwait(rows_a, sem_a, sem_a2)
            gather(c + 1, rows_b, sem_b, sem_b2)
            scat(c, rows_a)
            gwait(rows_b, sem_b, sem_b2)
            gather(c + 2, rows_a, sem_a, sem_a2)
            scat(c + 1, rows_b)
            return carry

        lax.fori_loop(0, (STEPS - 1) // 2, step, 0)
        gwait(rows_a, sem_a, sem_a2)
        scat(STEPS - 1, rows_a)
        plsc.subcore_barrier()

        for k in range(ZCOPIES):
            row = r0 + k * CHUNK
            pltpu.sync_copy(acc_sh.at[pl.ds(row, CHUNK)],
                            agg_out.at[cid, pl.ds(row, CHUNK)])

    return pl.kernel(body, out_type=out_type, mesh=_mesh,
                     scratch_types=scratch)


def _make_cnt_kernel():
    out_type = [jax.ShapeDtypeStruct((NC * N_PAD,), jnp.float32)]
    scratch = [
        pltpu.VMEM((STEPS, CHUNK), jnp.int32),
        pltpu.VMEM((N_PAD,), jnp.float32),
        pltpu.VMEM((ROWS_PER_TILE,), jnp.float32),
        pltpu.VMEM((ROWS_PER_TILE,), jnp.float32),
        pltpu.VMEM_SHARED((NS * N_PAD,), jnp.float32),
    ]

    def body(dst3_hbm, cnt_out, dst_big, hist_v, ctmp_v, cacc_v, cnt_sh):
        cid = lax.axis_index("c")
        sid = lax.axis_index("s")
        wid = cid * NS + sid
        r0 = sid * ROWS_PER_TILE

        pltpu.sync_copy(dst3_hbm.at[wid], dst_big)

        def zb(j, carry):
            hist_v[pl.ds(j * 16, 16)] = jnp.zeros((16,), jnp.float32)
            return carry
        lax.fori_loop(0, N_PAD // 16, zb, 0)

        ones16 = jnp.ones((16,), jnp.float32)

        def hb(m, carry):
            for k in range(CHUNK // 16):
                plsc.addupdate_scatter(
                    hist_v, [dst_big[m, pl.ds(k * 16, 16)]], ones16)
            return carry

        lax.fori_loop(0, STEPS, hb, 0)
        pltpu.sync_copy(hist_v, cnt_sh.at[pl.ds(sid * N_PAD, N_PAD)])
        plsc.subcore_barrier()

        for s in range(NS):
            pltpu.sync_copy(cnt_sh.at[pl.ds(s * N_PAD + r0, ROWS_PER_TILE)],
                            ctmp_v)

            def cb(j, carry, first=(s == 0)):
                sl = pl.ds(j * 16, 16)
                if first:
                    cacc_v[sl] = ctmp_v[sl]
                else:
                    cacc_v[sl] = cacc_v[sl] + ctmp_v[sl]
                return carry

            lax.fori_loop(0, ROWS_PER_TILE // 16, cb, 0)
        pltpu.sync_copy(
            cacc_v, cnt_out.at[pl.ds(cid * N_PAD + r0, ROWS_PER_TILE)])

    return pl.kernel(body, out_type=out_type, mesh=_mesh,
                     scratch_types=scratch,
                     compiler_params=pltpu.CompilerParams(
                         needs_layout_passes=False))


_edge_kernel = _make_edge_kernel()
_cnt_kernel = _make_cnt_kernel()


def _mean_matmul(agg_ref, c0_ref, c1_ref, h_ref, wl_ref, bl_ref, wr_ref):
    a = agg_ref[0] + agg_ref[1]
    inv = 1.0 / jnp.maximum(c0_ref[...] + c1_ref[...], 1.0)
    out = (jnp.dot(a * inv, wl_ref[...], preferred_element_type=jnp.float32)
           + bl_ref[...][None, :]
           + jnp.dot(h_ref[...], wr_ref[...], preferred_element_type=jnp.float32))
    return jnp.maximum(out, 0.0)


def _dense_body(agg_ref, c0_ref, c1_ref, h_ref, wl_ref, bl_ref, wr_ref, out_ref):
    out_ref[...] = _mean_matmul(agg_ref, c0_ref, c1_ref, h_ref, wl_ref, bl_ref,
                                wr_ref)


_DENSE_SPECS = [
    pl.BlockSpec((NC, RB, D), lambda i: (0, i, 0)),
    pl.BlockSpec((RB, 1), lambda i: (i, 0)),
    pl.BlockSpec((RB, 1), lambda i: (i, 0)),
    pl.BlockSpec((RB, D), lambda i: (i, 0)),
    pl.BlockSpec((D, D), lambda i: (0, 0)),
    pl.BlockSpec((D,), lambda i: (0,)),
    pl.BlockSpec((D, D), lambda i: (0, 0)),
]


def _dense(agg2, c0, c1, h, Wl, bl, Wr):
    return pl.pallas_call(
        _dense_body,
        grid=(NB,),
        in_specs=_DENSE_SPECS,
        out_specs=pl.BlockSpec((RB, D), lambda i: (i, 0)),
        out_shape=jax.ShapeDtypeStruct((N, D), jnp.float32),
    )(agg2, c0, c1, h, Wl, bl, Wr)


def _final_body(agg_ref, c0_ref, c1_ref, h_ref, wl_ref, bl_ref, wr_ref, batch_ref,
                wc1_ref, bc1_ref, wc2_ref, bc2_ref, out_ref, pool, cntg):
    i = pl.program_id(0)

    @pl.when(i == 0)
    def _init():
        pool[...] = jnp.zeros_like(pool)
        cntg[...] = jnp.zeros_like(cntg)

    h3 = _mean_matmul(agg_ref, c0_ref, c1_ref, h_ref, wl_ref, bl_ref, wr_ref)
    b = batch_ref[0]
    oh = (lax.broadcasted_iota(jnp.int32, (G, RB), 0)
          == jnp.broadcast_to(b, (G, RB))).astype(jnp.float32)
    pool[...] += jnp.dot(oh, h3, preferred_element_type=jnp.float32,
                         precision=lax.Precision.HIGHEST)
    cntg[...] += jnp.broadcast_to(jnp.sum(oh, axis=1, keepdims=True), (G, D))

    @pl.when(i == NB - 1)
    def _fin():
        g = pool[...] / jnp.maximum(cntg[...], 1.0)
        z = jnp.maximum(
            jnp.dot(g, wc1_ref[...], preferred_element_type=jnp.float32)
            + bc1_ref[...][None, :], 0.0)
        out_ref[...] = (jnp.dot(z, wc2_ref[...], preferred_element_type=jnp.float32)
                        + bc2_ref[...][None, :])


def _dense_final(agg2, c0, c1, h, Wl, bl, Wr, batch3, Wc1, bc1, Wc2, bc2):
    return pl.pallas_call(
        _final_body,
        grid=(NB,),
        in_specs=_DENSE_SPECS + [
            pl.BlockSpec((1, 1, RB), lambda i: (i, 0, 0)),
            pl.BlockSpec((D, D), lambda i: (0, 0)),
            pl.BlockSpec((D,), lambda i: (0,)),
            pl.BlockSpec((D, 2), lambda i: (0, 0)),
            pl.BlockSpec((2,), lambda i: (0,)),
        ],
        out_specs=pl.BlockSpec((G, 2), lambda i: (0, 0)),
        out_shape=jax.ShapeDtypeStruct((G, 2), jnp.float32),
        scratch_shapes=[
            pltpu.VMEM((G, D), jnp.float32),
            pltpu.VMEM((G, D), jnp.float32),
        ],
    )(agg2, c0, c1, h, Wl, bl, Wr, batch3, Wc1, bc1, Wc2, bc2)


def kernel(x, edge_index, batch, Wl0, bl0, Wr0, Wl1, bl1, Wr1, Wl2, bl2, Wr2,
           Wc1, bc1, Wc2, bc2):
    src = edge_index[0].reshape(NW, E_PER_W)
    dst = edge_index[1].reshape(NW, STEPS, CHUNK)
    zrows = jnp.zeros((CHUNK, D), jnp.float32)

    (cnt_flat,) = _cnt_kernel(dst)
    (agg0,) = _edge_kernel(x, src, dst, zrows)
    c0 = cnt_flat[:N_PAD].reshape(N_PAD, 1)
    c1 = cnt_flat[N_PAD:].reshape(N_PAD, 1)
    h1 = _dense(agg0, c0, c1, x, Wl0, bl0, Wr0)
    (agg1,) = _edge_kernel(h1, src, dst, zrows)
    h2 = _dense(agg1, c0, c1, h1, Wl1, bl1, Wr1)
    (agg2,) = _edge_kernel(h2, src, dst, zrows)
    batch3 = batch.reshape(NB, 1, RB)
    return _dense_final(agg2, c0, c1, h2, Wl2, bl2, Wr2, batch3,
                        Wc1, bc1, Wc2, bc2)


# speedup vs baseline: 1.4343x; 1.0005x over previous
import jax
import jax.numpy as jnp
from jax import lax
from jax.experimental import pallas as pl
from jax.experimental.pallas import tpu as pltpu
from jax.experimental.pallas import tpu_sc as plsc

N = 10000
E = 320000
D = 128
G = 64

NC = 2
NS = 16
NW = NC * NS
E_PER_W = E // NW
CHUNK = 80
STEPS = E_PER_W // CHUNK

N_PAD = 10240
ROWS_PER_TILE = N_PAD // NS
ZCOPIES = ROWS_PER_TILE // CHUNK

RB = 1000
NB = N // RB

_mesh = plsc.VectorSubcoreMesh(core_axis_name="c", subcore_axis_name="s")


def _make_edge_kernel():
    out_type = [jax.ShapeDtypeStruct((NC, N_PAD, D), jnp.float32)]
    scratch = [
        pltpu.VMEM((E_PER_W,), jnp.int32),
        pltpu.VMEM((STEPS, CHUNK), jnp.int32),
        pltpu.VMEM((CHUNK, D), jnp.float32),
        pltpu.VMEM((CHUNK, D), jnp.float32),
        pltpu.VMEM_SHARED((N_PAD, D), jnp.float32),
        pltpu.SemaphoreType.DMA,
        pltpu.SemaphoreType.DMA,
        pltpu.SemaphoreType.DMA,
        pltpu.SemaphoreType.DMA,
    ]

    def body(h_hbm, src2_hbm, dst3_hbm, zrows_hbm, agg_out,
             src_big, dst_big, rows_a, rows_b, acc_sh,
             sem_a, sem_a2, sem_b, sem_b2):
        cid = lax.axis_index("c")
        sid = lax.axis_index("s")
        wid = cid * NS + sid
        r0 = sid * ROWS_PER_TILE

        pltpu.sync_copy(src2_hbm.at[wid], src_big)
        pltpu.sync_copy(dst3_hbm.at[wid], dst_big)

        HC = CHUNK // 2

        def gather(c, buf, sem1, sem2):
            o = pl.multiple_of(c * CHUNK, 8)
            pltpu.async_copy(h_hbm.at[src_big.at[pl.ds(o, HC)]],
                             buf.at[pl.ds(0, HC)], sem1)
            pltpu.async_copy(h_hbm.at[src_big.at[pl.ds(o + HC, HC)]],
                             buf.at[pl.ds(HC, HC)], sem2)

        def gwait(buf, sem1, sem2):
            pltpu.make_async_copy(zrows_hbm.at[pl.ds(0, HC)],
                                  buf.at[pl.ds(0, HC)], sem1).wait()
            pltpu.make_async_copy(zrows_hbm.at[pl.ds(0, HC)],
                                  buf.at[pl.ds(HC, HC)], sem2).wait()

        def scat(c, buf):
            pltpu.sync_copy(buf, acc_sh.at[dst_big.at[c]], add=True)

        gather(0, rows_a, sem_a, sem_a2)

        pltpu.sync_copy(zrows_hbm, rows_b)
        for k in range(ZCOPIES):
            pltpu.sync_copy(rows_b, acc_sh.at[pl.ds(r0 + k * CHUNK, CHUNK)])
        plsc.subcore_barrier()

        def step(j, carry):
            c = 2 * j
            gwait(rows_a, sem_a, sem_a2)
            gather(c + 1, rows_b, sem_b, sem_b2)
            scat(c, rows_a)
            gwait(rows_b, sem_b, sem_b2)
            gather(c + 2, rows_a, sem_a, sem_a2)
            scat(c + 1, rows_b)
            return carry

        lax.fori_loop(0, (STEPS - 1) // 2, step, 0)
        gwait(rows_a, sem_a, sem_a2)
        scat(STEPS - 1, rows_a)
        plsc.subcore_barrier()

        for k in range(ZCOPIES):
            row = r0 + k * CHUNK
            pltpu.sync_copy(acc_sh.at[pl.ds(row, CHUNK)],
                            agg_out.at[cid, pl.ds(row, CHUNK)])

    return pl.kernel(body, out_type=out_type, mesh=_mesh,
                     scratch_types=scratch)


def _make_cnt_kernel():
    out_type = [jax.ShapeDtypeStruct((NC * N_PAD,), jnp.float32)]
    scratch = [
        pltpu.VMEM((STEPS, CHUNK), jnp.int32),
        pltpu.VMEM((N_PAD,), jnp.float32),
        pltpu.VMEM((ROWS_PER_TILE,), jnp.float32),
        pltpu.VMEM((ROWS_PER_TILE,), jnp.float32),
        pltpu.VMEM_SHARED((NS * N_PAD,), jnp.float32),
    ]

    def body(dst3_hbm, cnt_out, dst_big, hist_v, ctmp_v, cacc_v, cnt_sh):
        cid = lax.axis_index("c")
        sid = lax.axis_index("s")
        wid = cid * NS + sid
        r0 = sid * ROWS_PER_TILE

        pltpu.sync_copy(dst3_hbm.at[wid], dst_big)

        def zb(j, carry):
            hist_v[pl.ds(j * 16, 16)] = jnp.zeros((16,), jnp.float32)
            return carry
        lax.fori_loop(0, N_PAD // 16, zb, 0)

        ones16 = jnp.ones((16,), jnp.float32)

        def hb(m, carry):
            for k in range(CHUNK // 16):
                plsc.addupdate_scatter(
                    hist_v, [dst_big[m, pl.ds(k * 16, 16)]], ones16)
            return carry

        lax.fori_loop(0, STEPS, hb, 0)
        pltpu.sync_copy(hist_v, cnt_sh.at[pl.ds(sid * N_PAD, N_PAD)])
        plsc.subcore_barrier()

        for s in range(NS):
            pltpu.sync_copy(cnt_sh.at[pl.ds(s * N_PAD + r0, ROWS_PER_TILE)],
                            ctmp_v)

            def cb(j, carry, first=(s == 0)):
                sl = pl.ds(j * 16, 16)
                if first:
                    cacc_v[sl] = ctmp_v[sl]
                else:
                    cacc_v[sl] = cacc_v[sl] + ctmp_v[sl]
                return carry

            lax.fori_loop(0, ROWS_PER_TILE // 16, cb, 0)
        pltpu.sync_copy(
            cacc_v, cnt_out.at[pl.ds(cid * N_PAD + r0, ROWS_PER_TILE)])

    return pl.kernel(body, out_type=out_type, mesh=_mesh,
                     scratch_types=scratch,
                     compiler_params=pltpu.CompilerParams(
                         needs_layout_passes=False))


_edge_kernel = _make_edge_kernel()
_cnt_kernel = _make_cnt_kernel()


def _mean_matmul(agg_ref, c0_ref, c1_ref, h_ref, wl_ref, bl_ref, wr_ref):
    a = agg_ref[0] + agg_ref[1]
    inv = 1.0 / jnp.maximum(c0_ref[...] + c1_ref[...], 1.0)
    out = (jnp.dot(a * inv, wl_ref[...], preferred_element_type=jnp.float32)
           + bl_ref[...][None, :]
           + jnp.dot(h_ref[...], wr_ref[...], preferred_element_type=jnp.float32))
    return jnp.maximum(out, 0.0)


def _dense_body(agg_ref, c0_ref, c1_ref, h_ref, wl_ref, bl_ref, wr_ref, out_ref):
    out_ref[...] = _mean_matmul(agg_ref, c0_ref, c1_ref, h_ref, wl_ref, bl_ref,
                                wr_ref)


_DENSE_SPECS = [
    pl.BlockSpec((NC, RB, D), lambda i: (0, i, 0)),
    pl.BlockSpec((RB, 1), lambda i: (i, 0)),
    pl.BlockSpec((RB, 1), lambda i: (i, 0)),
    pl.BlockSpec((RB, D), lambda i: (i, 0)),
    pl.BlockSpec((D, D), lambda i: (0, 0)),
    pl.BlockSpec((D,), lambda i: (0,)),
    pl.BlockSpec((D, D), lambda i: (0, 0)),
]


def _dense(agg2, c0, c1, h, Wl, bl, Wr):
    return pl.pallas_call(
        _dense_body,
        grid=(NB,),
        in_specs=_DENSE_SPECS,
        out_specs=pl.BlockSpec((RB, D), lambda i: (i, 0)),
        out_shape=jax.ShapeDtypeStruct((N, D), jnp.float32),
    )(agg2, c0, c1, h, Wl, bl, Wr)


def _final_body(agg_ref, c0_ref, c1_ref, h_ref, wl_ref, bl_ref, wr_ref, batch_ref,
                wc1_ref, bc1_ref, wc2_ref, bc2_ref, out_ref, pool, cntg):
    i = pl.program_id(0)

    @pl.when(i == 0)
    def _init():
        pool[...] = jnp.zeros_like(pool)
        cntg[...] = jnp.zeros_like(cntg)

    h3 = _mean_matmul(agg_ref, c0_ref, c1_ref, h_ref, wl_ref, bl_ref, wr_ref)
    b = batch_ref[0]
    oh = (lax.broadcasted_iota(jnp.int32, (G, RB), 0)
          == jnp.broadcast_to(b, (G, RB))).astype(jnp.float32)
    pool[...] += jnp.dot(oh, h3, preferred_element_type=jnp.float32,
                         precision=lax.Precision.HIGHEST)
    cntg[...] += jnp.broadcast_to(jnp.sum(oh, axis=1, keepdims=True), (G, D))

    @pl.when(i == NB - 1)
    def _fin():
        g = pool[...] / jnp.maximum(cntg[...], 1.0)
        z = jnp.maximum(
            jnp.dot(g, wc1_ref[...], preferred_element_type=jnp.float32)
            + bc1_ref[...][None, :], 0.0)
        out_ref[...] = (jnp.dot(z, wc2_ref[...], preferred_element_type=jnp.float32)
                        + bc2_ref[...][None, :])


def _dense_final(agg2, c0, c1, h, Wl, bl, Wr, batch3, Wc1, bc1, Wc2, bc2):
    return pl.pallas_call(
        _final_body,
        grid=(NB,),
        in_specs=_DENSE_SPECS + [
            pl.BlockSpec((1, 1, RB), lambda i: (i, 0, 0)),
            pl.BlockSpec((D, D), lambda i: (0, 0)),
            pl.BlockSpec((D,), lambda i: (0,)),
            pl.BlockSpec((D, 2), lambda i: (0, 0)),
            pl.BlockSpec((2,), lambda i: (0,)),
        ],
        out_specs=pl.BlockSpec((G, 2), lambda i: (0, 0)),
        out_shape=jax.ShapeDtypeStruct((G, 2), jnp.float32),
        scratch_shapes=[
            pltpu.VMEM((G, D), jnp.float32),
            pltpu.VMEM((G, D), jnp.float32),
        ],
    )(agg2, c0, c1, h, Wl, bl, Wr, batch3, Wc1, bc1, Wc2, bc2)


def kernel(x, edge_index, batch, Wl0, bl0, Wr0, Wl1, bl1, Wr1, Wl2, bl2, Wr2,
           Wc1, bc1, Wc2, bc2):
    src = edge_index[0].reshape(NW, E_PER_W)
    dst = edge_index[1].reshape(NW, STEPS, CHUNK)
    zrows = jnp.zeros((CHUNK, D), jnp.float32)

    (cnt_flat,) = _cnt_kernel(dst)
    (agg0,) = _edge_kernel(x, src, dst, zrows)
    c0 = cnt_flat[:N_PAD].reshape(N_PAD, 1)
    c1 = cnt_flat[N_PAD:].reshape(N_PAD, 1)
    h1 = _dense(agg0, c0, c1, x, Wl0, bl0, Wr0)
    (agg1,) = _edge_kernel(h1, src, dst, zrows)
    h2 = _dense(agg1, c0, c1, h1, Wl1, bl1, Wr1)
    (agg2,) = _edge_kernel(h2, src, dst, zrows)
    batch3 = batch.reshape(NB, 1, RB)
    return _dense_final(agg2, c0, c1, h2, Wl2, bl2, Wr2, batch3,
                        Wc1, bc1, Wc2, bc2)


# speedup vs baseline: 1.4600x; 1.0179x over previous
import jax
import jax.numpy as jnp
from jax import lax
from jax.experimental import pallas as pl
from jax.experimental.pallas import tpu as pltpu
from jax.experimental.pallas import tpu_sc as plsc

N = 10000
E = 320000
D = 128
G = 64

NC = 2
NS = 16
NW = NC * NS
E_PER_W = E // NW
CHUNK = 80
STEPS = E_PER_W // CHUNK

N_PAD = 10240
ROWS_PER_TILE = N_PAD // NS
ZCOPIES = ROWS_PER_TILE // CHUNK

RB = 2000
NB = N // RB

_mesh = plsc.VectorSubcoreMesh(core_axis_name="c", subcore_axis_name="s")


def _make_edge_kernel():
    out_type = [jax.ShapeDtypeStruct((NC, N_PAD, D), jnp.float32)]
    scratch = [
        pltpu.VMEM((E_PER_W,), jnp.int32),
        pltpu.VMEM((STEPS, CHUNK), jnp.int32),
        pltpu.VMEM((CHUNK, D), jnp.float32),
        pltpu.VMEM((CHUNK, D), jnp.float32),
        pltpu.VMEM_SHARED((N_PAD, D), jnp.float32),
        pltpu.SemaphoreType.DMA,
        pltpu.SemaphoreType.DMA,
        pltpu.SemaphoreType.DMA,
        pltpu.SemaphoreType.DMA,
    ]

    def body(h_hbm, src2_hbm, dst3_hbm, zrows_hbm, agg_out,
             src_big, dst_big, rows_a, rows_b, acc_sh,
             sem_a, sem_a2, sem_b, sem_b2):
        cid = lax.axis_index("c")
        sid = lax.axis_index("s")
        wid = cid * NS + sid
        r0 = sid * ROWS_PER_TILE

        pltpu.sync_copy(src2_hbm.at[wid], src_big)
        pltpu.sync_copy(dst3_hbm.at[wid], dst_big)

        HC = CHUNK // 2

        def gather(c, buf, sem1, sem2):
            o = pl.multiple_of(c * CHUNK, 8)
            pltpu.async_copy(h_hbm.at[src_big.at[pl.ds(o, HC)]],
                             buf.at[pl.ds(0, HC)], sem1)
            pltpu.async_copy(h_hbm.at[src_big.at[pl.ds(o + HC, HC)]],
                             buf.at[pl.ds(HC, HC)], sem2)

        def gwait(buf, sem1, sem2):
            pltpu.make_async_copy(zrows_hbm.at[pl.ds(0, HC)],
                                  buf.at[pl.ds(0, HC)], sem1).wait()
            pltpu.make_async_copy(zrows_hbm.at[pl.ds(0, HC)],
                                  buf.at[pl.ds(HC, HC)], sem2).wait()

        def scat(c, buf):
            pltpu.sync_copy(buf, acc_sh.at[dst_big.at[c]], add=True)

        gather(0, rows_a, sem_a, sem_a2)

        pltpu.sync_copy(zrows_hbm, rows_b)
        for k in range(ZCOPIES):
            pltpu.sync_copy(rows_b, acc_sh.at[pl.ds(r0 + k * CHUNK, CHUNK)])
        plsc.subcore_barrier()

        def step(j, carry):
            c = 2 * j
            gwait(rows_a, sem_a, sem_a2)
            gather(c + 1, rows_b, sem_b, sem_b2)
            scat(c, rows_a)
            gwait(rows_b, sem_b, sem_b2)
            gather(c + 2, rows_a, sem_a, sem_a2)
            scat(c + 1, rows_b)
            return carry

        lax.fori_loop(0, (STEPS - 1) // 2, step, 0)
        gwait(rows_a, sem_a, sem_a2)
        scat(STEPS - 1, rows_a)
        plsc.subcore_barrier()

        for k in range(ZCOPIES):
            row = r0 + k * CHUNK
            pltpu.sync_copy(acc_sh.at[pl.ds(row, CHUNK)],
                            agg_out.at[cid, pl.ds(row, CHUNK)])

    return pl.kernel(body, out_type=out_type, mesh=_mesh,
                     scratch_types=scratch)


def _make_cnt_kernel():
    out_type = [jax.ShapeDtypeStruct((NC * N_PAD,), jnp.float32)]
    scratch = [
        pltpu.VMEM((STEPS, CHUNK), jnp.int32),
        pltpu.VMEM((N_PAD,), jnp.float32),
        pltpu.VMEM((ROWS_PER_TILE,), jnp.float32),
        pltpu.VMEM((ROWS_PER_TILE,), jnp.float32),
        pltpu.VMEM_SHARED((NS * N_PAD,), jnp.float32),
    ]

    def body(dst3_hbm, cnt_out, dst_big, hist_v, ctmp_v, cacc_v, cnt_sh):
        cid = lax.axis_index("c")
        sid = lax.axis_index("s")
        wid = cid * NS + sid
        r0 = sid * ROWS_PER_TILE

        pltpu.sync_copy(dst3_hbm.at[wid], dst_big)

        def zb(j, carry):
            hist_v[pl.ds(j * 16, 16)] = jnp.zeros((16,), jnp.float32)
            return carry
        lax.fori_loop(0, N_PAD // 16, zb, 0)

        ones16 = jnp.ones((16,), jnp.float32)

        def hb(m, carry):
            for k in range(CHUNK // 16):
                plsc.addupdate_scatter(
                    hist_v, [dst_big[m, pl.ds(k * 16, 16)]], ones16)
            return carry

        lax.fori_loop(0, STEPS, hb, 0)
        pltpu.sync_copy(hist_v, cnt_sh.at[pl.ds(sid * N_PAD, N_PAD)])
        plsc.subcore_barrier()

        for s in range(NS):
            pltpu.sync_copy(cnt_sh.at[pl.ds(s * N_PAD + r0, ROWS_PER_TILE)],
                            ctmp_v)

            def cb(j, carry, first=(s == 0)):
                sl = pl.ds(j * 16, 16)
                if first:
                    cacc_v[sl] = ctmp_v[sl]
                else:
                    cacc_v[sl] = cacc_v[sl] + ctmp_v[sl]
                return carry

            lax.fori_loop(0, ROWS_PER_TILE // 16, cb, 0)
        pltpu.sync_copy(
            cacc_v, cnt_out.at[pl.ds(cid * N_PAD + r0, ROWS_PER_TILE)])

    return pl.kernel(body, out_type=out_type, mesh=_mesh,
                     scratch_types=scratch,
                     compiler_params=pltpu.CompilerParams(
                         needs_layout_passes=False))


_edge_kernel = _make_edge_kernel()
_cnt_kernel = _make_cnt_kernel()


def _mean_matmul(agg_ref, c0_ref, c1_ref, h_ref, wl_ref, bl_ref, wr_ref):
    a = agg_ref[0] + agg_ref[1]
    inv = 1.0 / jnp.maximum(c0_ref[...] + c1_ref[...], 1.0)
    out = (jnp.dot(a * inv, wl_ref[...], preferred_element_type=jnp.float32)
           + bl_ref[...][None, :]
           + jnp.dot(h_ref[...], wr_ref[...], preferred_element_type=jnp.float32))
    return jnp.maximum(out, 0.0)


def _dense_body(agg_ref, c0_ref, c1_ref, h_ref, wl_ref, bl_ref, wr_ref, out_ref):
    out_ref[...] = _mean_matmul(agg_ref, c0_ref, c1_ref, h_ref, wl_ref, bl_ref,
                                wr_ref)


_DENSE_SPECS = [
    pl.BlockSpec((NC, RB, D), lambda i: (0, i, 0)),
    pl.BlockSpec((RB, 1), lambda i: (i, 0)),
    pl.BlockSpec((RB, 1), lambda i: (i, 0)),
    pl.BlockSpec((RB, D), lambda i: (i, 0)),
    pl.BlockSpec((D, D), lambda i: (0, 0)),
    pl.BlockSpec((D,), lambda i: (0,)),
    pl.BlockSpec((D, D), lambda i: (0, 0)),
]


def _dense(agg2, c0, c1, h, Wl, bl, Wr):
    return pl.pallas_call(
        _dense_body,
        grid=(NB,),
        in_specs=_DENSE_SPECS,
        out_specs=pl.BlockSpec((RB, D), lambda i: (i, 0)),
        out_shape=jax.ShapeDtypeStruct((N, D), jnp.float32),
    )(agg2, c0, c1, h, Wl, bl, Wr)


def _final_body(agg_ref, c0_ref, c1_ref, h_ref, wl_ref, bl_ref, wr_ref, batch_ref,
                wc1_ref, bc1_ref, wc2_ref, bc2_ref, out_ref, pool, cntg):
    i = pl.program_id(0)

    @pl.when(i == 0)
    def _init():
        pool[...] = jnp.zeros_like(pool)
        cntg[...] = jnp.zeros_like(cntg)

    h3 = _mean_matmul(agg_ref, c0_ref, c1_ref, h_ref, wl_ref, bl_ref, wr_ref)
    b = batch_ref[0]
    oh = (lax.broadcasted_iota(jnp.int32, (G, RB), 0)
          == jnp.broadcast_to(b, (G, RB))).astype(jnp.float32)
    pool[...] += jnp.dot(oh, h3, preferred_element_type=jnp.float32,
                         precision=lax.Precision.HIGHEST)
    cntg[...] += jnp.broadcast_to(jnp.sum(oh, axis=1, keepdims=True), (G, D))

    @pl.when(i == NB - 1)
    def _fin():
        g = pool[...] / jnp.maximum(cntg[...], 1.0)
        z = jnp.maximum(
            jnp.dot(g, wc1_ref[...], preferred_element_type=jnp.float32)
            + bc1_ref[...][None, :], 0.0)
        out_ref[...] = (jnp.dot(z, wc2_ref[...], preferred_element_type=jnp.float32)
                        + bc2_ref[...][None, :])


def _dense_final(agg2, c0, c1, h, Wl, bl, Wr, batch3, Wc1, bc1, Wc2, bc2):
    return pl.pallas_call(
        _final_body,
        grid=(NB,),
        in_specs=_DENSE_SPECS + [
            pl.BlockSpec((1, 1, RB), lambda i: (i, 0, 0)),
            pl.BlockSpec((D, D), lambda i: (0, 0)),
            pl.BlockSpec((D,), lambda i: (0,)),
            pl.BlockSpec((D, 2), lambda i: (0, 0)),
            pl.BlockSpec((2,), lambda i: (0,)),
        ],
        out_specs=pl.BlockSpec((G, 2), lambda i: (0, 0)),
        out_shape=jax.ShapeDtypeStruct((G, 2), jnp.float32),
        scratch_shapes=[
            pltpu.VMEM((G, D), jnp.float32),
            pltpu.VMEM((G, D), jnp.float32),
        ],
    )(agg2, c0, c1, h, Wl, bl, Wr, batch3, Wc1, bc1, Wc2, bc2)


def kernel(x, edge_index, batch, Wl0, bl0, Wr0, Wl1, bl1, Wr1, Wl2, bl2, Wr2,
           Wc1, bc1, Wc2, bc2):
    src = edge_index[0].reshape(NW, E_PER_W)
    dst = edge_index[1].reshape(NW, STEPS, CHUNK)
    zrows = jnp.zeros((CHUNK, D), jnp.float32)

    (cnt_flat,) = _cnt_kernel(dst)
    (agg0,) = _edge_kernel(x, src, dst, zrows)
    c0 = cnt_flat[:N_PAD].reshape(N_PAD, 1)
    c1 = cnt_flat[N_PAD:].reshape(N_PAD, 1)
    h1 = _dense(agg0, c0, c1, x, Wl0, bl0, Wr0)
    (agg1,) = _edge_kernel(h1, src, dst, zrows)
    h2 = _dense(agg1, c0, c1, h1, Wl1, bl1, Wr1)
    (agg2,) = _edge_kernel(h2, src, dst, zrows)
    batch3 = batch.reshape(NB, 1, RB)
    return _dense_final(agg2, c0, c1, h2, Wl2, bl2, Wr2, batch3,
                        Wc1, bc1, Wc2, bc2)
